# baseline (speedup 1.0000x reference)
import jax
import jax.numpy as jnp
from jax import lax
from jax.experimental import pallas as pl
from jax.experimental.pallas import tpu as pltpu
from jax.experimental.pallas import tpu_sc as plsc

N_POINTS = 1048576
NCORES = 2
NSUB = 16
NW = NCORES * NSUB
PW = N_POINTS // NW
P = 256
HC = 2
NB = PW // P
NB2 = NB // 2
L = 16

RES = (128, 256, 512, 1024)
GRID_DEFS = tuple((a, b, R) for (a, b) in ((0, 1), (0, 2), (1, 2)) for R in RES)
NG = len(GRID_DEFS)
NF = 2
NOUT = NG * NF


def _sc_body(xflat, *rest):
    tables = rest[:NG]
    out = rest[NG]
    (coords, idxb, wgt, dstb, acc,
     gsem0, gsem1, csem0, csem1, osem0, osem1) = rest[NG + 1:]
    gsem = (gsem0, gsem1)
    csem = (csem0, csem1)
    osem = (osem0, osem1)

    wid = lax.axis_index("s") * NCORES + lax.axis_index("c")
    iot = lax.iota(jnp.int32, L)
    pat_pt = lax.shift_right_logical(iot, 1)
    pat_f = lax.bitwise_and(iot, 1)
    pat_x3 = iot * 3
    maxb = jnp.int32(NB - 1)

    def fire_coords(s, b):
        base = wid * PW + jnp.minimum(b, maxb) * P
        return pltpu.async_copy(
            xflat.at[pl.ds(base * 3, P * 3)], coords.at[s], csem[s])

    def load(s, b):
        pltpu.make_async_copy(
            xflat.at[pl.ds(0, P * 3)], coords.at[s], csem[s]).wait()
        for g, (ca, cb, R) in enumerate(GRID_DEFS):
            def idx_body(j, c, s=s, g=g, ca=ca, cb=cb, R=R):
                sl = pl.ds(j * L, L)
                xo = j * (3 * L)
                u = plsc.load_gather(coords.at[s], [pat_x3 + (xo + ca)])
                v = plsc.load_gather(coords.at[s], [pat_x3 + (xo + cb)])
                pu = u * jnp.float32(R - 1)
                pv = v * jnp.float32(R - 1)
                r0 = pu.astype(jnp.int32)
                c0 = pv.astype(jnp.int32)
                idxb[s, g, j // (128 // L), pl.ds((j % (128 // L)) * L, L)] = (
                    r0 * R + c0)
                wgt[s, 2 * g + 0, sl] = pu - r0.astype(jnp.float32)
                wgt[s, 2 * g + 1, sl] = pv - c0.astype(jnp.float32)
                return c

            lax.fori_loop(0, P // L, idx_body, 0)
            for h in range(HC):
                pltpu.async_copy(
                    tables[g].at[idxb.at[s, g, h]],
                    dstb.at[s, g, pl.ds(h * 128, 128)], gsem[s])
        fire_coords(s, b + 2)

    def wait_gathers(s):
        for g in range(NG):
            for h in range(HC):
                pltpu.make_async_copy(
                    tables[g].at[idxb.at[s, g, h]],
                    dstb.at[s, g, pl.ds(h * 128, 128)], gsem[s]).wait()

    def wait_out(s):
        pltpu.make_async_copy(
            acc.at[s], out.at[pl.ds(0, P * NOUT)], osem[s]).wait()

    def interp(s, b):
        base = wid * PW + b * P
        for g in range(NG):
            gsp = jnp.full((L,), g, jnp.int32)
            cp0 = pat_f
            cp1 = pat_f + 2
            cp2 = pat_f + 4
            cp3 = pat_f + 6
            pat_acc = pat_pt * NOUT + pat_f + 2 * g
            wr = jnp.full((L,), 2 * g + 0, jnp.int32)
            wc = jnp.full((L,), 2 * g + 1, jnp.int32)

            def interp_body(j, c, s=s, gsp=gsp, pat_acc=pat_acc, wr=wr,
                            wc=wc, cp0=cp0, cp1=cp1, cp2=cp2, cp3=cp3):
                pt = pat_pt + j * (L // 2)
                g00 = plsc.load_gather(dstb.at[s], [gsp, pt, cp0])
                g01 = plsc.load_gather(dstb.at[s], [gsp, pt, cp1])
                g10 = plsc.load_gather(dstb.at[s], [gsp, pt, cp2])
                g11 = plsc.load_gather(dstb.at[s], [gsp, pt, cp3])
                fr = plsc.load_gather(wgt.at[s], [wr, pt])
                fc = plsc.load_gather(wgt.at[s], [wc, pt])
                h0 = g00 + fc * (g01 - g00)
                h1 = g10 + fc * (g11 - g10)
                res = h0 + fr * (h1 - h0)
                plsc.store_scatter(acc.at[s],
                                   [pat_acc + j * ((L // 2) * NOUT)], res)
                return c

            lax.fori_loop(0, (P * NF) // L, interp_body, 0)
        pltpu.async_copy(acc.at[s], out.at[pl.ds(base * NOUT, P * NOUT)],
                         osem[s])

    fire_coords(0, jnp.int32(0))
    fire_coords(1, jnp.int32(1))
    load(0, jnp.int32(0))

    def pair_body(bb, carry):
        b0 = bb * 2
        load(1, b0 + 1)
        wait_gathers(0)

        @pl.when(bb > 0)
        def _():
            wait_out(0)

        interp(0, b0)
        load(0, b0 + 2)
        wait_gathers(1)

        @pl.when(bb > 0)
        def _():
            wait_out(1)

        interp(1, b0 + 1)
        return carry

    lax.fori_loop(0, NB2, pair_body, 0)

    wait_gathers(0)
    pltpu.make_async_copy(
        xflat.at[pl.ds(0, P * 3)], coords.at[0], csem[0]).wait()
    pltpu.make_async_copy(
        xflat.at[pl.ds(0, P * 3)], coords.at[1], csem[1]).wait()
    wait_out(0)
    wait_out(1)


def _pack_t8(g, R):
    rr = R * R
    t = g.reshape(rr, NF)
    tp = jnp.concatenate([t, jnp.zeros((R + 1, NF), jnp.float32)], axis=0)
    return jnp.concatenate(
        [tp[:rr], tp[1:rr + 1], tp[R:rr + R], tp[R + 1:rr + R + 1]], axis=1)


def kernel(x, bound, xy_g0, xy_g1, xy_g2, xy_g3,
           yz_g0, yz_g1, yz_g2, yz_g3,
           xz_g0, xz_g1, xz_g2, xz_g3):
    del bound
    grids = (xy_g0, xy_g1, xy_g2, xy_g3,
             yz_g0, yz_g1, yz_g2, yz_g3,
             xz_g0, xz_g1, xz_g2, xz_g3)
    tabs = [_pack_t8(g, R) for g, (_, _, R) in zip(grids, GRID_DEFS)]

    f = pl.kernel(
        _sc_body,
        out_type=jax.ShapeDtypeStruct((N_POINTS * NOUT,), jnp.float32),
        mesh=plsc.VectorSubcoreMesh(
            core_axis_name="c", subcore_axis_name="s",
            num_cores=NCORES, num_subcores=NSUB),
        scratch_types=[
            pltpu.VMEM((2, 3 * P), jnp.float32),
            pltpu.VMEM((2, NG, HC, 128), jnp.int32),
            pltpu.VMEM((2, 2 * NG, P), jnp.float32),
            pltpu.VMEM((2, NG, P, 8), jnp.float32),
            pltpu.VMEM((2, P * NOUT), jnp.float32),
            pltpu.SemaphoreType.DMA,
            pltpu.SemaphoreType.DMA,
            pltpu.SemaphoreType.DMA,
            pltpu.SemaphoreType.DMA,
            pltpu.SemaphoreType.DMA,
            pltpu.SemaphoreType.DMA,
        ],
        compiler_params=pltpu.CompilerParams(
            needs_layout_passes=False, use_tc_tiling_on_sc=False),
    )
    return f(x.reshape(-1), *tabs).reshape(N_POINTS, NOUT)


# speedup vs baseline: 1.3185x; 1.3185x over previous
import jax
import jax.numpy as jnp
from jax import lax
from jax.experimental import pallas as pl
from jax.experimental.pallas import tpu as pltpu
from jax.experimental.pallas import tpu_sc as plsc

N_POINTS = 1048576
NCORES = 2
NSUB = 16
NW = NCORES * NSUB
PW = N_POINTS // NW
P = 256
HC = 2
NB = PW // P
NB2 = NB // 2
L = 16

RES = (128, 256, 512, 1024)
GRID_DEFS = tuple((a, b, R) for (a, b) in ((0, 1), (0, 2), (1, 2)) for R in RES)
NG = len(GRID_DEFS)
NF = 2
NOUT = NG * NF


def _sc_body(xflat, *rest):
    tables = rest[:NG]
    out = rest[NG]
    (coords, idxb, wgt, dstb, acc,
     gsem0, gsem1, csem0, csem1, osem0, osem1) = rest[NG + 1:]
    gsem = (gsem0, gsem1)
    csem = (csem0, csem1)
    osem = (osem0, osem1)

    wid = lax.axis_index("s") * NCORES + lax.axis_index("c")
    iot = lax.iota(jnp.int32, L)
    pat_pt = lax.shift_right_logical(iot, 1)
    pat_f = lax.bitwise_and(iot, 1)
    maxb = jnp.int32(NB - 1)

    def fire_coords(s, b):
        base = wid * PW + jnp.minimum(b, maxb) * P
        return pltpu.async_copy(
            xflat.at[:, pl.ds(base, P)], coords.at[s], csem[s])

    def load(s, b):
        pltpu.make_async_copy(
            xflat.at[:, pl.ds(0, P)], coords.at[s], csem[s]).wait()
        for g, (ca, cb, R) in enumerate(GRID_DEFS):
            def idx_body(j, c, s=s, g=g, ca=ca, cb=cb, R=R):
                sl = pl.ds(j * L, L)
                u = coords[s, ca, sl]
                v = coords[s, cb, sl]
                pu = u * jnp.float32(R - 1)
                pv = v * jnp.float32(R - 1)
                r0 = pu.astype(jnp.int32)
                c0 = pv.astype(jnp.int32)
                idxb[s, g, j // (128 // L), pl.ds((j % (128 // L)) * L, L)] = (
                    r0 * R + c0)
                wgt[s, 2 * g + 0, sl] = pu - r0.astype(jnp.float32)
                wgt[s, 2 * g + 1, sl] = pv - c0.astype(jnp.float32)
                return c

            lax.fori_loop(0, P // L, idx_body, 0)
            for h in range(HC):
                pltpu.async_copy(
                    tables[g].at[idxb.at[s, g, h]],
                    dstb.at[s, g, pl.ds(h * 128, 128)], gsem[s])
        fire_coords(s, b + 2)

    def wait_gathers(s):
        for g in range(NG):
            for h in range(HC):
                pltpu.make_async_copy(
                    tables[g].at[idxb.at[s, g, h]],
                    dstb.at[s, g, pl.ds(h * 128, 128)], gsem[s]).wait()

    def wait_out(s):
        pltpu.make_async_copy(
            acc.at[s], out.at[pl.ds(0, P * NOUT)], osem[s]).wait()

    def interp(s, b):
        base = wid * PW + b * P
        for g in range(NG):
            gsp = jnp.full((L,), g, jnp.int32)
            cp0 = pat_f
            cp1 = pat_f + 2
            cp2 = pat_f + 4
            cp3 = pat_f + 6
            pat_acc = pat_pt * NOUT + pat_f + 2 * g
            wr = jnp.full((L,), 2 * g + 0, jnp.int32)
            wc = jnp.full((L,), 2 * g + 1, jnp.int32)

            def interp_body(j, c, s=s, gsp=gsp, pat_acc=pat_acc, wr=wr,
                            wc=wc, cp0=cp0, cp1=cp1, cp2=cp2, cp3=cp3):
                pt = pat_pt + j * (L // 2)
                g00 = plsc.load_gather(dstb.at[s], [gsp, pt, cp0])
                g01 = plsc.load_gather(dstb.at[s], [gsp, pt, cp1])
                g10 = plsc.load_gather(dstb.at[s], [gsp, pt, cp2])
                g11 = plsc.load_gather(dstb.at[s], [gsp, pt, cp3])
                fr = plsc.load_gather(wgt.at[s], [wr, pt])
                fc = plsc.load_gather(wgt.at[s], [wc, pt])
                h0 = g00 + fc * (g01 - g00)
                h1 = g10 + fc * (g11 - g10)
                res = h0 + fr * (h1 - h0)
                plsc.store_scatter(acc.at[s],
                                   [pat_acc + j * ((L // 2) * NOUT)], res)
                return c

            lax.fori_loop(0, (P * NF) // L, interp_body, 0)
        pltpu.async_copy(acc.at[s], out.at[pl.ds(base * NOUT, P * NOUT)],
                         osem[s])

    fire_coords(0, jnp.int32(0))
    fire_coords(1, jnp.int32(1))
    load(0, jnp.int32(0))

    def pair_body(bb, carry):
        b0 = bb * 2
        load(1, b0 + 1)
        wait_gathers(0)

        @pl.when(bb > 0)
        def _():
            wait_out(0)

        interp(0, b0)
        load(0, b0 + 2)
        wait_gathers(1)

        @pl.when(bb > 0)
        def _():
            wait_out(1)

        interp(1, b0 + 1)
        return carry

    lax.fori_loop(0, NB2, pair_body, 0)

    wait_gathers(0)
    pltpu.make_async_copy(
        xflat.at[:, pl.ds(0, P)], coords.at[0], csem[0]).wait()
    pltpu.make_async_copy(
        xflat.at[:, pl.ds(0, P)], coords.at[1], csem[1]).wait()
    wait_out(0)
    wait_out(1)


def _pack_t8(g, R):
    rr = R * R
    t = g.reshape(rr, NF)
    tp = jnp.concatenate([t, jnp.zeros((R + 1, NF), jnp.float32)], axis=0)
    return jnp.concatenate(
        [tp[:rr], tp[1:rr + 1], tp[R:rr + R], tp[R + 1:rr + R + 1]], axis=1)


def kernel(x, bound, xy_g0, xy_g1, xy_g2, xy_g3,
           yz_g0, yz_g1, yz_g2, yz_g3,
           xz_g0, xz_g1, xz_g2, xz_g3):
    del bound
    grids = (xy_g0, xy_g1, xy_g2, xy_g3,
             yz_g0, yz_g1, yz_g2, yz_g3,
             xz_g0, xz_g1, xz_g2, xz_g3)
    tabs = [_pack_t8(g, R) for g, (_, _, R) in zip(grids, GRID_DEFS)]

    f = pl.kernel(
        _sc_body,
        out_type=jax.ShapeDtypeStruct((N_POINTS * NOUT,), jnp.float32),
        mesh=plsc.VectorSubcoreMesh(
            core_axis_name="c", subcore_axis_name="s",
            num_cores=NCORES, num_subcores=NSUB),
        scratch_types=[
            pltpu.VMEM((2, 3, P), jnp.float32),
            pltpu.VMEM((2, NG, HC, 128), jnp.int32),
            pltpu.VMEM((2, 2 * NG, P), jnp.float32),
            pltpu.VMEM((2, NG, P, 8), jnp.float32),
            pltpu.VMEM((2, P * NOUT), jnp.float32),
            pltpu.SemaphoreType.DMA,
            pltpu.SemaphoreType.DMA,
            pltpu.SemaphoreType.DMA,
            pltpu.SemaphoreType.DMA,
            pltpu.SemaphoreType.DMA,
            pltpu.SemaphoreType.DMA,
        ],
        compiler_params=pltpu.CompilerParams(
            needs_layout_passes=False, use_tc_tiling_on_sc=False),
    )
    return f(x.T, *tabs).reshape(N_POINTS, NOUT)


# speedup vs baseline: 1.4809x; 1.1231x over previous
import jax
import jax.numpy as jnp
from jax import lax
from jax.experimental import pallas as pl
from jax.experimental.pallas import tpu as pltpu
from jax.experimental.pallas import tpu_sc as plsc

N_POINTS = 1048576
NCORES = 2
NSUB = 16
NW = NCORES * NSUB
PW = N_POINTS // NW
P = 256
HC = 2
NB = PW // P
NB2 = NB // 2
L = 16

RES = (128, 256, 512, 1024)
GRID_DEFS = tuple((a, b, R) for (a, b) in ((0, 1), (0, 2), (1, 2)) for R in RES)
NG = len(GRID_DEFS)
NF = 2
NOUT = NG * NF


def _sc_body(xflat, *rest):
    tables = rest[:NG]
    out = rest[NG]
    (coords, idxb, wgt, dstb, acc,
     gsem0, gsem1, csem0, csem1, osem0, osem1) = rest[NG + 1:]
    gsem = (gsem0, gsem1)
    csem = (csem0, csem1)
    osem = (osem0, osem1)

    wid = lax.axis_index("s") * NCORES + lax.axis_index("c")
    iot = lax.iota(jnp.int32, L)
    pat_pt = lax.shift_right_logical(iot, 1)
    pat_f = lax.bitwise_and(iot, 1)
    maxb = jnp.int32(NB - 1)

    def fire_coords(s, b):
        base = wid * PW + jnp.minimum(b, maxb) * P
        return pltpu.async_copy(
            xflat.at[:, pl.ds(base, P)], coords.at[s], csem[s])

    def load(s, b):
        pltpu.make_async_copy(
            xflat.at[:, pl.ds(0, P)], coords.at[s], csem[s]).wait()
        for g, (ca, cb, R) in enumerate(GRID_DEFS):
            @plsc.parallel_loop(0, P // L, 1, unroll=2)
            def idx_body(j, s=s, g=g, ca=ca, cb=cb, R=R):
                sl = pl.ds(j * L, L)
                u = coords[s, ca, sl]
                v = coords[s, cb, sl]
                pu = u * jnp.float32(R - 1)
                pv = v * jnp.float32(R - 1)
                r0 = pu.astype(jnp.int32)
                c0 = pv.astype(jnp.int32)
                idxb[s, g, j // (128 // L), pl.ds((j % (128 // L)) * L, L)] = (
                    r0 * R + c0)
                wgt[s, 2 * g + 0, sl] = pu - r0.astype(jnp.float32)
                wgt[s, 2 * g + 1, sl] = pv - c0.astype(jnp.float32)
            for h in range(HC):
                pltpu.async_copy(
                    tables[g].at[idxb.at[s, g, h]],
                    dstb.at[s, g, pl.ds(h * 128, 128)], gsem[s])
        fire_coords(s, b + 2)

    def wait_gathers(s):
        for g in range(NG):
            for h in range(HC):
                pltpu.make_async_copy(
                    tables[g].at[idxb.at[s, g, h]],
                    dstb.at[s, g, pl.ds(h * 128, 128)], gsem[s]).wait()

    def wait_out(s):
        pltpu.make_async_copy(
            acc.at[s], out.at[pl.ds(0, P * NOUT)], osem[s]).wait()

    def interp(s, b):
        base = wid * PW + b * P
        for g in range(NG):
            gsp = jnp.full((L,), g, jnp.int32)
            cp0 = pat_f
            cp1 = pat_f + 2
            cp2 = pat_f + 4
            cp3 = pat_f + 6
            pat_acc = pat_pt * NOUT + pat_f + 2 * g
            wr = jnp.full((L,), 2 * g + 0, jnp.int32)
            wc = jnp.full((L,), 2 * g + 1, jnp.int32)

            @plsc.parallel_loop(0, (P * NF) // L, 1, unroll=4)
            def interp_body(j, s=s, gsp=gsp, pat_acc=pat_acc, wr=wr,
                            wc=wc, cp0=cp0, cp1=cp1, cp2=cp2, cp3=cp3):
                pt = pat_pt + j * (L // 2)
                g00 = plsc.load_gather(dstb.at[s], [gsp, pt, cp0])
                g01 = plsc.load_gather(dstb.at[s], [gsp, pt, cp1])
                g10 = plsc.load_gather(dstb.at[s], [gsp, pt, cp2])
                g11 = plsc.load_gather(dstb.at[s], [gsp, pt, cp3])
                fr = plsc.load_gather(wgt.at[s], [wr, pt])
                fc = plsc.load_gather(wgt.at[s], [wc, pt])
                h0 = g00 + fc * (g01 - g00)
                h1 = g10 + fc * (g11 - g10)
                res = h0 + fr * (h1 - h0)
                plsc.store_scatter(acc.at[s],
                                   [pat_acc + j * ((L // 2) * NOUT)], res)
        pltpu.async_copy(acc.at[s], out.at[pl.ds(base * NOUT, P * NOUT)],
                         osem[s])

    fire_coords(0, jnp.int32(0))
    fire_coords(1, jnp.int32(1))
    load(0, jnp.int32(0))

    def pair_body(bb, carry):
        b0 = bb * 2
        load(1, b0 + 1)
        wait_gathers(0)

        @pl.when(bb > 0)
        def _():
            wait_out(0)

        interp(0, b0)
        load(0, b0 + 2)
        wait_gathers(1)

        @pl.when(bb > 0)
        def _():
            wait_out(1)

        interp(1, b0 + 1)
        return carry

    lax.fori_loop(0, NB2, pair_body, 0)

    wait_gathers(0)
    pltpu.make_async_copy(
        xflat.at[:, pl.ds(0, P)], coords.at[0], csem[0]).wait()
    pltpu.make_async_copy(
        xflat.at[:, pl.ds(0, P)], coords.at[1], csem[1]).wait()
    wait_out(0)
    wait_out(1)


def _pack_t8(g, R):
    rr = R * R
    t = g.reshape(rr, NF)
    tp = jnp.concatenate([t, jnp.zeros((R + 1, NF), jnp.float32)], axis=0)
    return jnp.concatenate(
        [tp[:rr], tp[1:rr + 1], tp[R:rr + R], tp[R + 1:rr + R + 1]], axis=1)


def kernel(x, bound, xy_g0, xy_g1, xy_g2, xy_g3,
           yz_g0, yz_g1, yz_g2, yz_g3,
           xz_g0, xz_g1, xz_g2, xz_g3):
    del bound
    grids = (xy_g0, xy_g1, xy_g2, xy_g3,
             yz_g0, yz_g1, yz_g2, yz_g3,
             xz_g0, xz_g1, xz_g2, xz_g3)
    tabs = [_pack_t8(g, R) for g, (_, _, R) in zip(grids, GRID_DEFS)]

    f = pl.kernel(
        _sc_body,
        out_type=jax.ShapeDtypeStruct((N_POINTS * NOUT,), jnp.float32),
        mesh=plsc.VectorSubcoreMesh(
            core_axis_name="c", subcore_axis_name="s",
            num_cores=NCORES, num_subcores=NSUB),
        scratch_types=[
            pltpu.VMEM((2, 3, P), jnp.float32),
            pltpu.VMEM((2, NG, HC, 128), jnp.int32),
            pltpu.VMEM((2, 2 * NG, P), jnp.float32),
            pltpu.VMEM((2, NG, P, 8), jnp.float32),
            pltpu.VMEM((2, P * NOUT), jnp.float32),
            pltpu.SemaphoreType.DMA,
            pltpu.SemaphoreType.DMA,
            pltpu.SemaphoreType.DMA,
            pltpu.SemaphoreType.DMA,
            pltpu.SemaphoreType.DMA,
            pltpu.SemaphoreType.DMA,
        ],
        compiler_params=pltpu.CompilerParams(
            needs_layout_passes=False, use_tc_tiling_on_sc=False),
    )
    return f(x.T, *tabs).reshape(N_POINTS, NOUT)
